# baseline (speedup 1.0000x reference)
import functools

import jax
import jax.numpy as jnp
from jax import lax
from jax.experimental import pallas as pl
from jax.experimental.pallas import tpu as pltpu
from jax.experimental.pallas import tpu_sc as plsc

NC = 2
NS = 16
LN = 16
K = 128
NPT1 = 640
PADN = 8

_F32 = jnp.float32


def _l2n(v):
    n = jnp.sqrt(jnp.sum(v * v, axis=-1, keepdims=True))
    return v / jnp.maximum(n, 1e-12)


def _tc_ab(x, W1, b1):
    N, C = x.shape
    H = C // NC
    BN = 1000

    def body(x_ref, w_ref, b_ref, a_ref, bb_ref):
        xb = x_ref[...]
        a = jnp.dot(xb, w_ref[0:C, :],
                    preferred_element_type=_F32) + b_ref[...]
        bb = jnp.dot(xb, w_ref[C:2 * C, :],
                     preferred_element_type=_F32)
        a_ref[0] = a[:, 0:H]
        a_ref[1] = a[:, H:C]
        bb_ref[0] = bb[:, 0:H]
        bb_ref[1] = bb[:, H:C]

    return pl.pallas_call(
        body,
        grid=(N // BN,),
        in_specs=[pl.BlockSpec((BN, C), lambda i: (i, 0)),
                  pl.BlockSpec((2 * C, C), lambda i: (0, 0)),
                  pl.BlockSpec((1, C), lambda i: (0, 0))],
        out_specs=[pl.BlockSpec((NC, BN, H), lambda i: (0, i, 0))] * 2,
        out_shape=[jax.ShapeDtypeStruct((NC, N + PADN, H), _F32)] * 2,
    )(x, W1, b1.reshape(1, C))


def _tc_msg(P3, b2):
    rows = P3.shape[1]
    BR = 256

    def body(p0_ref, p1_ref, b_ref, m_ref):
        v = (p0_ref[0] + p1_ref[0]).reshape(BR, K, LN)
        s = jnp.sum(v, axis=-1) + b_ref[...]
        m_ref[...] = 1.0 / (1.0 + jnp.exp(-s))

    return pl.pallas_call(
        body,
        grid=(rows // BR,),
        in_specs=[pl.BlockSpec((1, BR, K * LN), lambda i: (0, i, 0)),
                  pl.BlockSpec((1, BR, K * LN), lambda i: (1, i, 0)),
                  pl.BlockSpec((1, 1), lambda i: (0, 0))],
        out_specs=[pl.BlockSpec((BR, K), lambda i: (i, 0))],
        out_shape=[jax.ShapeDtypeStruct((rows, K), _F32)],
    )(P3, P3, b2.reshape(1, 1))[0]


def _tc_deg(cnt1, x):
    N = x.shape[0]
    BN = 1000

    def body(c_ref, cnt_ref, di_ref):
        cnt = c_ref[...]
        cnt_ref[...] = cnt
        di_ref[...] = lax.rsqrt(jnp.maximum(cnt + 1.0, 1.0))

    return pl.pallas_call(
        body,
        grid=(N // BN,),
        in_specs=[pl.BlockSpec((BN, 1), lambda i: (i, 0))],
        out_specs=[pl.BlockSpec((BN, 1), lambda i: (i, 0)),
                   pl.BlockSpec((BN, 1), lambda i: (i, 0))],
        out_shape=[jax.ShapeDtypeStruct((N, 1), _F32),
                   jax.ShapeDtypeStruct((N, 1), _F32)],
    )(cnt1)


def _tc_xs(dinv, xt):
    N, C = xt.shape
    H = C // NC
    BN = 1000

    def body(di_ref, x_ref, xs_ref):
        xs = di_ref[...] * x_ref[...]
        xs_ref[0] = xs[:, 0:H]
        xs_ref[1] = xs[:, H:C]

    return pl.pallas_call(
        body,
        grid=(N // BN,),
        in_specs=[pl.BlockSpec((BN, 1), lambda i: (i, 0)),
                  pl.BlockSpec((BN, C), lambda i: (i, 0))],
        out_specs=[pl.BlockSpec((NC, BN, H), lambda i: (0, i, 0))],
        out_shape=[jax.ShapeDtypeStruct((NC, N + PADN, H), _F32)],
    )(dinv, xt)[0]


def _split_mm(h0, h1, w_ref, C, H):
    return (jnp.dot(h0, w_ref[0:H, :], preferred_element_type=_F32)
            + jnp.dot(h1, w_ref[H:C, :], preferred_element_type=_F32))


def _tc_layer0(acc, sums1, cnt, xt, dinv, lw, hw):
    N, C = xt.shape
    H = C // NC
    BN = 1000

    def body(a0_ref, a1_ref, s_ref, c_ref, xt_ref, di_ref, lw_ref, hw_ref,
             lh_ref, xn_ref):
        lh = s_ref[...] / jnp.maximum(c_ref[...], 1.0)
        lh_ref[...] = lh
        di = di_ref[...]
        xt = xt_ref[...]
        xLh = (di * (a0_ref[0] + di * xt[:, 0:H]),
               di * (a1_ref[0] + di * xt[:, H:C]))
        xHh = (xt[:, 0:H] - xLh[0], xt[:, H:C] - xLh[1])
        xL = jnp.maximum(_l2n(_split_mm(xLh[0], xLh[1], lw_ref, C, H)), 0.0)
        xH = jnp.maximum(_l2n(_split_mm(xHh[0], xHh[1], hw_ref, C, H)), 0.0)
        xn_ref[...] = lh * xL + (1.0 - lh) * xH

    return pl.pallas_call(
        body,
        grid=(N // BN,),
        in_specs=[pl.BlockSpec((1, BN, H), lambda i: (0, i, 0)),
                  pl.BlockSpec((1, BN, H), lambda i: (1, i, 0)),
                  pl.BlockSpec((BN, 1), lambda i: (i, 0)),
                  pl.BlockSpec((BN, 1), lambda i: (i, 0)),
                  pl.BlockSpec((BN, C), lambda i: (i, 0)),
                  pl.BlockSpec((BN, 1), lambda i: (i, 0)),
                  pl.BlockSpec((C, C), lambda i: (0, 0)),
                  pl.BlockSpec((C, C), lambda i: (0, 0))],
        out_specs=[pl.BlockSpec((BN, 1), lambda i: (i, 0)),
                   pl.BlockSpec((BN, C), lambda i: (i, 0))],
        out_shape=[jax.ShapeDtypeStruct((N, 1), _F32),
                   jax.ShapeDtypeStruct((N, C), _F32)],
    )(acc, acc, sums1, cnt, xt, dinv, lw, hw)


def _tc_final(acc, xt, lh, dinv, lw, hw, ow, ob):
    N, C = xt.shape
    H = C // NC
    OUT = ow.shape[1]
    BN = 1000

    def body(a0_ref, a1_ref, xt_ref, lh_ref, di_ref, lw_ref, hw_ref,
             ow_ref, ob_ref, o_ref):
        di = di_ref[...]
        xt = xt_ref[...]
        xLh = (di * (a0_ref[0] + di * xt[:, 0:H]),
               di * (a1_ref[0] + di * xt[:, H:C]))
        xHh = (xt[:, 0:H] - xLh[0], xt[:, H:C] - xLh[1])
        xL = jnp.maximum(_l2n(_split_mm(xLh[0], xLh[1], lw_ref, C, H)), 0.0)
        xH = jnp.maximum(_l2n(_split_mm(xHh[0], xHh[1], hw_ref, C, H)), 0.0)
        lh = lh_ref[...]
        xn = _l2n(lh * xL + (1.0 - lh) * xH)
        o_ref[...] = jnp.dot(xn, ow_ref[...],
                             preferred_element_type=_F32) + ob_ref[...]

    return pl.pallas_call(
        body,
        grid=(N // BN,),
        in_specs=[pl.BlockSpec((1, BN, H), lambda i: (0, i, 0)),
                  pl.BlockSpec((1, BN, H), lambda i: (1, i, 0)),
                  pl.BlockSpec((BN, C), lambda i: (i, 0)),
                  pl.BlockSpec((BN, 1), lambda i: (i, 0)),
                  pl.BlockSpec((BN, 1), lambda i: (i, 0)),
                  pl.BlockSpec((C, C), lambda i: (0, 0)),
                  pl.BlockSpec((C, C), lambda i: (0, 0)),
                  pl.BlockSpec((C, OUT), lambda i: (0, 0)),
                  pl.BlockSpec((1, OUT), lambda i: (0, 0))],
        out_specs=[pl.BlockSpec((BN, OUT), lambda i: (i, 0))],
        out_shape=[jax.ShapeDtypeStruct((N, OUT), _F32)],
    )(acc, acc, xt, lh, dinv, lw, hw, ow, ob.reshape(1, OUT))[0]


def _mesh():
    return plsc.VectorSubcoreMesh(core_axis_name="c", subcore_axis_name="s",
                                  num_cores=NC, num_subcores=NS)


def _sc_edge_mlp(Ast, Bst, srco, dsto, w2st):
    H = Ast.shape[1]
    CPT = srco.shape[1] // NS
    Ep = srco.shape[1] * K
    NTS = NS * NPT1

    @functools.partial(
        pl.kernel,
        out_type=[jax.ShapeDtypeStruct((NS, NPT1), _F32),
                  jax.ShapeDtypeStruct((NC, Ep, LN), _F32)],
        mesh=_mesh(),
        compiler_params=pltpu.CompilerParams(use_tc_tiling_on_sc=False),
        scratch_types=[
            pltpu.VMEM((CPT, K), jnp.int32),
            pltpu.VMEM((CPT, K), jnp.int32),
            pltpu.VMEM((K, H), _F32),
            pltpu.VMEM((K, H), _F32),
            pltpu.VMEM((K, H), _F32),
            pltpu.VMEM((K, H), _F32),
            pltpu.VMEM((K, LN), _F32),
            pltpu.VMEM((K, LN), _F32),
            pltpu.VMEM((K,), _F32),
            pltpu.VMEM((H,), _F32),
            pltpu.VMEM((NPT1,), _F32),
            pltpu.VMEM_SHARED((NTS,), _F32),
            pltpu.SemaphoreType.DMA,
            pltpu.SemaphoreType.DMA,
            pltpu.SemaphoreType.DMA,
            pltpu.SemaphoreType.DMA,
            pltpu.SemaphoreType.DMA,
            pltpu.SemaphoreType.DMA,
            pltpu.SemaphoreType.DMA,
            pltpu.SemaphoreType.DMA,
        ],
    )
    def body(a_hbm, b_hbm, src_hbm, dst_hbm, w2_hbm, cnt_hbm, p_hbm,
             sidx, didx, ga0, ga1, gb0, gb1, po0, po1, onesb, w2v, stage,
             cnt_sh, sa0, sa1, sb0, sb1, sp0, sp1, sc0, sc1):
        cc = lax.axis_index("c")
        ss = lax.axis_index("s")
        pltpu.sync_copy(src_hbm.at[cc, pl.ds(ss * CPT, CPT)], sidx)
        pltpu.sync_copy(dst_hbm.at[cc, pl.ds(ss * CPT, CPT)], didx)
        pltpu.sync_copy(w2_hbm.at[cc], w2v)

        onef = jnp.ones((LN,), _F32)
        zerof = jnp.zeros((LN,), _F32)
        for i in range(K // LN):
            onesb[pl.ds(i * LN, LN)] = onef
        for i in range(NPT1 // LN):
            stage[pl.ds(i * LN, LN)] = zerof
        pltpu.sync_copy(stage, cnt_sh.at[pl.ds(ss * NPT1, NPT1)])
        plsc.subcore_barrier()

        gas = (ga0, ga1)
        gbs = (gb0, gb1)
        pos = (po0, po1)
        sas = (sa0, sa1)
        sbs = (sb0, sb1)
        sps = (sp0, sp1)
        sos = (sc0, sc1)

        w2c = tuple(w2v[pl.ds(k2 * LN, LN)] for k2 in range(H // LN))
        ebase = ss * CPT * K

        def start(c, b):
            pltpu.async_copy(a_hbm.at[didx.at[c]], gas[b], sas[b])
            pltpu.async_copy(b_hbm.at[sidx.at[c]], gbs[b], sbs[b])

        start(0, 0)

        def chunk(i, _):
            for b in range(2):
                c = i * 2 + b
                pltpu.make_async_copy(a_hbm.at[didx.at[c]], gas[b],
                                      sas[b]).wait()
                pltpu.make_async_copy(b_hbm.at[sidx.at[c]], gbs[b],
                                      sbs[b]).wait()

                @pl.when(c + 1 < CPT)
                def _():
                    start(c + 1, 1 - b)

                @pl.when(c >= 2)
                def _():
                    pltpu.make_async_copy(
                        pos[b],
                        p_hbm.at[cc, pl.ds(ebase + (c - 2) * K, K)],
                        sps[b]).wait()

                ga, gb, po = gas[b], gbs[b], pos[b]

                @plsc.parallel_loop(0, K // LN, unroll=2)
                def _(g):
                    for j in range(LN):
                        row = g * LN + j
                        p = zerof
                        for k2 in range(H // LN):
                            sl = pl.ds(k2 * LN, LN)
                            t = ga[row, sl] + gb[row, sl]
                            p = p + (t * w2c[k2]) / (1.0 + jnp.exp(-t))
                        po[row, pl.ds(0, LN)] = p
                pltpu.async_copy(
                    po, p_hbm.at[cc, pl.ds(ebase + c * K, K)], sps[b])

                @pl.when(cc == 0)
                def _():
                    @pl.when(c >= 2)
                    def _():
                        pltpu.make_async_copy(
                            onesb, cnt_sh.at[didx.at[c - 2]],
                            sos[b]).wait()
                    pltpu.async_copy(onesb, cnt_sh.at[didx.at[c]],
                                     sos[b], add=True)
            return 0

        lax.fori_loop(0, CPT // 2, chunk, 0)

        @pl.when(cc == 0)
        def _():
            for b2 in range(2):
                pltpu.make_async_copy(
                    onesb, cnt_sh.at[didx.at[CPT - 2 + b2]],
                    sos[b2]).wait()
        for b in range(2):
            c = CPT - 2 + b
            pltpu.make_async_copy(
                pos[b], p_hbm.at[cc, pl.ds(ebase + c * K, K)],
                sps[b]).wait()
        plsc.subcore_barrier()

        @pl.when(cc == 0)
        def _():
            pltpu.sync_copy(cnt_sh.at[pl.ds(ss * NPT1, NPT1)], stage)
            pltpu.sync_copy(stage, cnt_hbm.at[ss])

    return body(Ast, Bst, srco, dsto, w2st)


def _sc_prop(xst, srco, dstp, zc, msg=None):
    H = xst.shape[1]
    CPT = dstp.shape[0] // NS
    NTS = NS * NPT1
    ZR = K
    ZCH = NPT1 // ZR

    NB = 4
    out_type = [jax.ShapeDtypeStruct((NC, NS, NPT1, H), _F32)]
    scratch = (
        [pltpu.VMEM((CPT, K), jnp.int32),
         pltpu.VMEM((CPT, K), jnp.int32)]
        + [pltpu.VMEM((K, H), _F32)] * NB
        + [pltpu.VMEM_SHARED((NTS, H), _F32)]
        + [pltpu.SemaphoreType.DMA] * (2 * NB)
    )
    if msg is not None:
        out_type.append(jax.ShapeDtypeStruct((NS, NPT1), _F32))
        scratch += ([pltpu.VMEM((K,), _F32)] * NB
                    + [pltpu.VMEM((NPT1,), _F32),
                       pltpu.VMEM_SHARED((NTS,), _F32)]
                    + [pltpu.SemaphoreType.DMA] * (2 * NB))

    @functools.partial(
        pl.kernel, out_type=out_type, mesh=_mesh(),
        compiler_params=pltpu.CompilerParams(use_tc_tiling_on_sc=False),
        scratch_types=scratch)
    def body(xs_hbm, src_hbm, dst_hbm, z_hbm, *rest):
        if msg is not None:
            (msg_hbm, acc_hbm, sum_hbm, sidx, didx) = rest[:5]
            bufs = rest[5:5 + NB]
            acc_sh = rest[5 + NB]
            sems = rest[6 + NB:6 + 2 * NB]
            scs = rest[6 + 2 * NB:6 + 3 * NB]
            msgvr = rest[6 + 3 * NB:6 + 4 * NB]
            stage, sums_sh = rest[6 + 4 * NB:8 + 4 * NB]
            sms = rest[8 + 4 * NB:8 + 5 * NB]
            smg = rest[8 + 5 * NB:8 + 6 * NB]
        else:
            (acc_hbm, sidx, didx) = rest[:3]
            bufs = rest[3:3 + NB]
            acc_sh = rest[3 + NB]
            sems = rest[4 + NB:4 + 2 * NB]
            scs = rest[4 + 2 * NB:4 + 3 * NB]
        buf0 = bufs[0]
        cc = lax.axis_index("c")
        ss = lax.axis_index("s")
        pltpu.sync_copy(src_hbm.at[cc, pl.ds(ss * CPT, CPT)], sidx)
        pltpu.sync_copy(dst_hbm.at[pl.ds(ss * CPT, CPT)], didx)
        for z in range(ZCH):
            pltpu.sync_copy(z_hbm.at[ss, pl.ds(z * ZR, ZR)], buf0)
            pltpu.sync_copy(buf0,
                            acc_sh.at[pl.ds(ss * NPT1 + z * ZR, ZR)])
        if msg is not None:
            zerof = jnp.zeros((LN,), _F32)
            for i in range(NPT1 // LN):
                stage[pl.ds(i * LN, LN)] = zerof
            pltpu.sync_copy(stage, sums_sh.at[pl.ds(ss * NPT1, NPT1)])
        plsc.subcore_barrier()

        def start(c, b):
            pltpu.async_copy(xs_hbm.at[sidx.at[c]], bufs[b], sems[b])
            if msg is not None:
                @pl.when(cc == 0)
                def _():
                    pltpu.async_copy(
                        msg_hbm.at[pl.ds((ss * CPT + c) * K, K)],
                        msgvr[b], smg[b])

        def wait_scatter(c, b):
            pltpu.make_async_copy(bufs[b], acc_sh.at[didx.at[c]],
                                  scs[b]).wait()
            if msg is not None:
                @pl.when(cc == 0)
                def _():
                    pltpu.make_async_copy(msgvr[b],
                                          sums_sh.at[didx.at[c]],
                                          sms[b]).wait()

        start(0, 0)
        start(1, 1)

        def chunk(i, _):
            for b in range(NB):
                c = i * NB + b
                pltpu.make_async_copy(xs_hbm.at[sidx.at[c]], bufs[b],
                                      sems[b]).wait()

                @pl.when(c >= 2)
                def _():
                    wait_scatter(c - 2, (b - 2) % NB)

                @pl.when(c + 2 < CPT)
                def _():
                    start(c + 2, (b + 2) % NB)

                pltpu.async_copy(bufs[b], acc_sh.at[didx.at[c]], scs[b],
                                 add=True)
                if msg is not None:
                    @pl.when(cc == 0)
                    def _():
                        pltpu.make_async_copy(
                            msg_hbm.at[pl.ds((ss * CPT + c) * K, K)],
                            msgvr[b], smg[b]).wait()
                        pltpu.async_copy(msgvr[b],
                                         sums_sh.at[didx.at[c]],
                                         sms[b], add=True)
            return 0

        lax.fori_loop(0, CPT // NB, chunk, 0)
        wait_scatter(CPT - 2, (CPT - 2) % NB)
        wait_scatter(CPT - 1, (CPT - 1) % NB)
        plsc.subcore_barrier()
        for z in range(ZCH):
            pltpu.sync_copy(acc_sh.at[pl.ds(ss * NPT1 + z * ZR, ZR)], buf0)
            pltpu.sync_copy(buf0, acc_hbm.at[cc, ss, pl.ds(z * ZR, ZR)])
        if msg is not None:
            @pl.when(cc == 0)
            def _():
                pltpu.sync_copy(sums_sh.at[pl.ds(ss * NPT1, NPT1)], stage)
                pltpu.sync_copy(stage, sum_hbm.at[ss])

    if msg is not None:
        return body(xst, srco, dstp, zc, msg)
    return body(xst, srco, dstp, zc)


def kernel(x, edge_index, mlp_W1, mlp_b1, mlp_W2, mlp_b2,
           low_W0, high_W0, low_W1, high_W1, out_W, out_b):
    N, C = x.shape
    H = C // NC
    E = edge_index.shape[1]
    N1 = N + PADN
    cpt = -(-E // (NS * K * 8)) * 8
    epad = NS * K * cpt
    pad = epad - E
    src = jnp.concatenate([edge_index[0], jnp.full((pad,), N, jnp.int32)])
    dst = jnp.concatenate([edge_index[1], jnp.full((pad,), N, jnp.int32)])
    off = jnp.arange(NC, dtype=jnp.int32)[:, None, None] * N1
    src2o = src.reshape(1, -1, K) + off
    dst2o = dst.reshape(1, -1, K) + off
    dst2p = dst.reshape(-1, K)

    Ast, Bst = _tc_ab(x, mlp_W1, mlp_b1)
    w2st = mlp_W2[:, 0].reshape(NC, H)
    cnt_parts, P = _sc_edge_mlp(Ast.reshape(NC * N1, H),
                                Bst.reshape(NC * N1, H), src2o, dst2o, w2st)
    cnt, dinv = _tc_deg(cnt_parts.reshape(-1, 1), x)
    msg = _tc_msg(P.reshape(NC, -1, K * LN), mlp_b2)

    zc = jnp.zeros((NS, NPT1, H), _F32)

    xs0st = _tc_xs(dinv, x)
    acc0, sums_parts = _sc_prop(xs0st.reshape(NC * N1, H), src2o, dst2p, zc,
                                msg=msg.reshape(-1))
    lh, x1 = _tc_layer0(acc0.reshape(NC, -1, H), sums_parts.reshape(-1, 1),
                        cnt, x, dinv, low_W0, high_W0)
    xs1st = _tc_xs(dinv, x1)
    acc1 = _sc_prop(xs1st.reshape(NC * N1, H), src2o, dst2p, zc)[0]
    return _tc_final(acc1.reshape(NC, -1, H), x1, lh, dinv,
                     low_W1, high_W1, out_W, out_b)


# speedup vs baseline: 1.7103x; 1.7103x over previous
import functools

import jax
import jax.numpy as jnp
from jax import lax
from jax.experimental import pallas as pl
from jax.experimental.pallas import tpu as pltpu
from jax.experimental.pallas import tpu_sc as plsc

NC = 2
NS = 16
LN = 16
K = 128
NPT1 = 640
PADN = 8

_F32 = jnp.float32


def _l2n(v):
    n = jnp.sqrt(jnp.sum(v * v, axis=-1, keepdims=True))
    return v / jnp.maximum(n, 1e-12)


def _tc_ab(x, W1, b1):
    N, C = x.shape
    H = C // NC
    BN = 1000

    def body(x_ref, w_ref, b_ref, a_ref, bb_ref):
        xb = x_ref[...]
        a = jnp.dot(xb, w_ref[0:C, :],
                    preferred_element_type=_F32) + b_ref[...]
        bb = jnp.dot(xb, w_ref[C:2 * C, :],
                     preferred_element_type=_F32)
        a_ref[0] = a[:, 0:H]
        a_ref[1] = a[:, H:C]
        bb_ref[0] = bb[:, 0:H]
        bb_ref[1] = bb[:, H:C]

    return pl.pallas_call(
        body,
        grid=(N // BN,),
        in_specs=[pl.BlockSpec((BN, C), lambda i: (i, 0)),
                  pl.BlockSpec((2 * C, C), lambda i: (0, 0)),
                  pl.BlockSpec((1, C), lambda i: (0, 0))],
        out_specs=[pl.BlockSpec((NC, BN, H), lambda i: (0, i, 0))] * 2,
        out_shape=[jax.ShapeDtypeStruct((NC, N + PADN, H), _F32)] * 2,
    )(x, W1, b1.reshape(1, C))


def _tc_msg(P3, b2):
    rows = P3.shape[1]
    BR = 256

    def body(p0_ref, p1_ref, b_ref, m_ref):
        v = (p0_ref[0] + p1_ref[0]).reshape(BR, K, LN)
        s = jnp.sum(v, axis=-1) + b_ref[...]
        m_ref[...] = 1.0 / (1.0 + jnp.exp(-s))

    return pl.pallas_call(
        body,
        grid=(rows // BR,),
        in_specs=[pl.BlockSpec((1, BR, K * LN), lambda i: (0, i, 0)),
                  pl.BlockSpec((1, BR, K * LN), lambda i: (1, i, 0)),
                  pl.BlockSpec((1, 1), lambda i: (0, 0))],
        out_specs=[pl.BlockSpec((BR, K), lambda i: (i, 0))],
        out_shape=[jax.ShapeDtypeStruct((rows, K), _F32)],
    )(P3, P3, b2.reshape(1, 1))[0]


def _tc_deg(cnt1, x):
    N = x.shape[0]
    BN = 1000

    def body(c_ref, cnt_ref, di_ref):
        cnt = c_ref[...]
        cnt_ref[...] = cnt
        di_ref[...] = lax.rsqrt(jnp.maximum(cnt + 1.0, 1.0))

    return pl.pallas_call(
        body,
        grid=(N // BN,),
        in_specs=[pl.BlockSpec((BN, 1), lambda i: (i, 0))],
        out_specs=[pl.BlockSpec((BN, 1), lambda i: (i, 0)),
                   pl.BlockSpec((BN, 1), lambda i: (i, 0))],
        out_shape=[jax.ShapeDtypeStruct((N, 1), _F32),
                   jax.ShapeDtypeStruct((N, 1), _F32)],
    )(cnt1)


def _tc_xs(dinv, xt):
    N, C = xt.shape
    H = C // NC
    BN = 1000

    def body(di_ref, x_ref, xs_ref):
        xs = di_ref[...] * x_ref[...]
        xs_ref[0] = xs[:, 0:H]
        xs_ref[1] = xs[:, H:C]

    return pl.pallas_call(
        body,
        grid=(N // BN,),
        in_specs=[pl.BlockSpec((BN, 1), lambda i: (i, 0)),
                  pl.BlockSpec((BN, C), lambda i: (i, 0))],
        out_specs=[pl.BlockSpec((NC, BN, H), lambda i: (0, i, 0))],
        out_shape=[jax.ShapeDtypeStruct((NC, N + PADN, H), _F32)],
    )(dinv, xt)[0]


def _split_mm(h0, h1, w_ref, C, H):
    return (jnp.dot(h0, w_ref[0:H, :], preferred_element_type=_F32)
            + jnp.dot(h1, w_ref[H:C, :], preferred_element_type=_F32))


def _tc_layer0(acc, sums1, cnt, xt, dinv, lw, hw):
    N, C = xt.shape
    H = C // NC
    BN = 1000

    def body(a0_ref, a1_ref, s_ref, c_ref, xt_ref, di_ref, lw_ref, hw_ref,
             lh_ref, xn_ref):
        lh = s_ref[...] / jnp.maximum(c_ref[...], 1.0)
        lh_ref[...] = lh
        di = di_ref[...]
        xt = xt_ref[...]
        xLh = (di * (a0_ref[0] + di * xt[:, 0:H]),
               di * (a1_ref[0] + di * xt[:, H:C]))
        xHh = (xt[:, 0:H] - xLh[0], xt[:, H:C] - xLh[1])
        xL = jnp.maximum(_l2n(_split_mm(xLh[0], xLh[1], lw_ref, C, H)), 0.0)
        xH = jnp.maximum(_l2n(_split_mm(xHh[0], xHh[1], hw_ref, C, H)), 0.0)
        xn_ref[...] = lh * xL + (1.0 - lh) * xH

    return pl.pallas_call(
        body,
        grid=(N // BN,),
        in_specs=[pl.BlockSpec((1, BN, H), lambda i: (0, i, 0)),
                  pl.BlockSpec((1, BN, H), lambda i: (1, i, 0)),
                  pl.BlockSpec((BN, 1), lambda i: (i, 0)),
                  pl.BlockSpec((BN, 1), lambda i: (i, 0)),
                  pl.BlockSpec((BN, C), lambda i: (i, 0)),
                  pl.BlockSpec((BN, 1), lambda i: (i, 0)),
                  pl.BlockSpec((C, C), lambda i: (0, 0)),
                  pl.BlockSpec((C, C), lambda i: (0, 0))],
        out_specs=[pl.BlockSpec((BN, 1), lambda i: (i, 0)),
                   pl.BlockSpec((BN, C), lambda i: (i, 0))],
        out_shape=[jax.ShapeDtypeStruct((N, 1), _F32),
                   jax.ShapeDtypeStruct((N, C), _F32)],
    )(acc, acc, sums1, cnt, xt, dinv, lw, hw)


def _tc_final(acc, xt, lh, dinv, lw, hw, ow, ob):
    N, C = xt.shape
    H = C // NC
    OUT = ow.shape[1]
    BN = 1000

    def body(a0_ref, a1_ref, xt_ref, lh_ref, di_ref, lw_ref, hw_ref,
             ow_ref, ob_ref, o_ref):
        di = di_ref[...]
        xt = xt_ref[...]
        xLh = (di * (a0_ref[0] + di * xt[:, 0:H]),
               di * (a1_ref[0] + di * xt[:, H:C]))
        xHh = (xt[:, 0:H] - xLh[0], xt[:, H:C] - xLh[1])
        xL = jnp.maximum(_l2n(_split_mm(xLh[0], xLh[1], lw_ref, C, H)), 0.0)
        xH = jnp.maximum(_l2n(_split_mm(xHh[0], xHh[1], hw_ref, C, H)), 0.0)
        lh = lh_ref[...]
        xn = _l2n(lh * xL + (1.0 - lh) * xH)
        o_ref[...] = jnp.dot(xn, ow_ref[...],
                             preferred_element_type=_F32) + ob_ref[...]

    return pl.pallas_call(
        body,
        grid=(N // BN,),
        in_specs=[pl.BlockSpec((1, BN, H), lambda i: (0, i, 0)),
                  pl.BlockSpec((1, BN, H), lambda i: (1, i, 0)),
                  pl.BlockSpec((BN, C), lambda i: (i, 0)),
                  pl.BlockSpec((BN, 1), lambda i: (i, 0)),
                  pl.BlockSpec((BN, 1), lambda i: (i, 0)),
                  pl.BlockSpec((C, C), lambda i: (0, 0)),
                  pl.BlockSpec((C, C), lambda i: (0, 0)),
                  pl.BlockSpec((C, OUT), lambda i: (0, 0)),
                  pl.BlockSpec((1, OUT), lambda i: (0, 0))],
        out_specs=[pl.BlockSpec((BN, OUT), lambda i: (i, 0))],
        out_shape=[jax.ShapeDtypeStruct((N, OUT), _F32)],
    )(acc, acc, xt, lh, dinv, lw, hw, ow, ob.reshape(1, OUT))[0]


def _mesh():
    return plsc.VectorSubcoreMesh(core_axis_name="c", subcore_axis_name="s",
                                  num_cores=NC, num_subcores=NS)


def _sc_edge_mlp(Ast, Bst, srco, dsto, w2st):
    H = Ast.shape[1]
    CPT = srco.shape[1] // NS
    Ep = srco.shape[1] * K
    NTS = NS * NPT1

    @functools.partial(
        pl.kernel,
        out_type=[jax.ShapeDtypeStruct((NS, NPT1), _F32),
                  jax.ShapeDtypeStruct((NC, Ep, LN), _F32)],
        mesh=_mesh(),
        compiler_params=pltpu.CompilerParams(use_tc_tiling_on_sc=False),
        scratch_types=[
            pltpu.VMEM((CPT, K), jnp.int32),
            pltpu.VMEM((CPT, K), jnp.int32),
            pltpu.VMEM((K, H), _F32),
            pltpu.VMEM((K, H), _F32),
            pltpu.VMEM((K, H), _F32),
            pltpu.VMEM((K, H), _F32),
            pltpu.VMEM((K, LN), _F32),
            pltpu.VMEM((K, LN), _F32),
            pltpu.VMEM((K,), _F32),
            pltpu.VMEM((H,), _F32),
            pltpu.VMEM((NPT1,), _F32),
            pltpu.VMEM_SHARED((NTS,), _F32),
            pltpu.SemaphoreType.DMA,
            pltpu.SemaphoreType.DMA,
            pltpu.SemaphoreType.DMA,
            pltpu.SemaphoreType.DMA,
            pltpu.SemaphoreType.DMA,
            pltpu.SemaphoreType.DMA,
            pltpu.SemaphoreType.DMA,
            pltpu.SemaphoreType.DMA,
        ],
    )
    def body(a_hbm, b_hbm, src_hbm, dst_hbm, w2_hbm, cnt_hbm, p_hbm,
             sidx, didx, ga0, ga1, gb0, gb1, po0, po1, onesb, w2v, stage,
             cnt_sh, sa0, sa1, sb0, sb1, sp0, sp1, sc0, sc1):
        cc = lax.axis_index("c")
        ss = lax.axis_index("s")
        pltpu.sync_copy(src_hbm.at[cc, pl.ds(ss * CPT, CPT)], sidx)
        pltpu.sync_copy(dst_hbm.at[cc, pl.ds(ss * CPT, CPT)], didx)
        pltpu.sync_copy(w2_hbm.at[cc], w2v)

        onef = jnp.ones((LN,), _F32)
        zerof = jnp.zeros((LN,), _F32)
        for i in range(K // LN):
            onesb[pl.ds(i * LN, LN)] = onef
        for i in range(NPT1 // LN):
            stage[pl.ds(i * LN, LN)] = zerof
        pltpu.sync_copy(stage, cnt_sh.at[pl.ds(ss * NPT1, NPT1)])
        plsc.subcore_barrier()

        gas = (ga0, ga1)
        gbs = (gb0, gb1)
        pos = (po0, po1)
        sas = (sa0, sa1)
        sbs = (sb0, sb1)
        sps = (sp0, sp1)
        sos = (sc0, sc1)

        w2c = tuple(w2v[pl.ds(k2 * LN, LN)] for k2 in range(H // LN))
        ebase = ss * CPT * K

        def start(c, b):
            pltpu.async_copy(a_hbm.at[didx.at[c]], gas[b], sas[b])
            pltpu.async_copy(b_hbm.at[sidx.at[c]], gbs[b], sbs[b])

        start(0, 0)

        def chunk(i, _):
            for b in range(2):
                c = i * 2 + b
                pltpu.make_async_copy(a_hbm.at[didx.at[c]], gas[b],
                                      sas[b]).wait()
                pltpu.make_async_copy(b_hbm.at[sidx.at[c]], gbs[b],
                                      sbs[b]).wait()

                @pl.when(c + 1 < CPT)
                def _():
                    start(c + 1, 1 - b)

                @pl.when(c >= 2)
                def _():
                    pltpu.make_async_copy(
                        pos[b],
                        p_hbm.at[cc, pl.ds(ebase + (c - 2) * K, K)],
                        sps[b]).wait()

                ga, gb, po = gas[b], gbs[b], pos[b]

                def group(g, _g):
                    for j in range(LN):
                        row = g * LN + j
                        us = []
                        ds = []
                        for k2 in range(H // LN):
                            sl = pl.ds(k2 * LN, LN)
                            t = ga[row, sl] + gb[row, sl]
                            us.append(t * w2c[k2])
                            tc = jnp.minimum(jnp.maximum(t, -21.0), 21.0)
                            ds.append(1.0 + jnp.exp(-tc))
                        d01 = ds[0] * ds[1]
                        d23 = ds[2] * ds[3]
                        n01 = (us[0] * ds[1] + us[1] * ds[0]) * d23
                        n23 = (us[2] * ds[3] + us[3] * ds[2]) * d01
                        po[row, pl.ds(0, LN)] = (n01 + n23) / (d01 * d23)
                    return 0

                lax.fori_loop(0, K // LN, group, 0)
                pltpu.async_copy(
                    po, p_hbm.at[cc, pl.ds(ebase + c * K, K)], sps[b])

                @pl.when(cc == 0)
                def _():
                    @pl.when(c >= 2)
                    def _():
                        pltpu.make_async_copy(
                            onesb, cnt_sh.at[didx.at[c - 2]],
                            sos[b]).wait()
                    pltpu.async_copy(onesb, cnt_sh.at[didx.at[c]],
                                     sos[b], add=True)
            return 0

        lax.fori_loop(0, CPT // 2, chunk, 0)

        @pl.when(cc == 0)
        def _():
            for b2 in range(2):
                pltpu.make_async_copy(
                    onesb, cnt_sh.at[didx.at[CPT - 2 + b2]],
                    sos[b2]).wait()
        for b in range(2):
            c = CPT - 2 + b
            pltpu.make_async_copy(
                pos[b], p_hbm.at[cc, pl.ds(ebase + c * K, K)],
                sps[b]).wait()
        plsc.subcore_barrier()

        @pl.when(cc == 0)
        def _():
            pltpu.sync_copy(cnt_sh.at[pl.ds(ss * NPT1, NPT1)], stage)
            pltpu.sync_copy(stage, cnt_hbm.at[ss])

    return body(Ast, Bst, srco, dsto, w2st)


def _sc_prop(xst, srco, dstp, zc, msg=None):
    H = xst.shape[1]
    CPT = dstp.shape[0] // NS
    NTS = NS * NPT1
    ZR = K
    ZCH = NPT1 // ZR

    NB = 4
    out_type = [jax.ShapeDtypeStruct((NC, NS, NPT1, H), _F32)]
    scratch = (
        [pltpu.VMEM((CPT, K), jnp.int32),
         pltpu.VMEM((CPT, K), jnp.int32)]
        + [pltpu.VMEM((K, H), _F32)] * NB
        + [pltpu.VMEM_SHARED((NTS, H), _F32)]
        + [pltpu.SemaphoreType.DMA] * (2 * NB)
    )
    if msg is not None:
        out_type.append(jax.ShapeDtypeStruct((NS, NPT1), _F32))
        scratch += ([pltpu.VMEM((K,), _F32)] * NB
                    + [pltpu.VMEM((NPT1,), _F32),
                       pltpu.VMEM_SHARED((NTS,), _F32)]
                    + [pltpu.SemaphoreType.DMA] * (2 * NB))

    @functools.partial(
        pl.kernel, out_type=out_type, mesh=_mesh(),
        compiler_params=pltpu.CompilerParams(use_tc_tiling_on_sc=False),
        scratch_types=scratch)
    def body(xs_hbm, src_hbm, dst_hbm, z_hbm, *rest):
        if msg is not None:
            (msg_hbm, acc_hbm, sum_hbm, sidx, didx) = rest[:5]
            bufs = rest[5:5 + NB]
            acc_sh = rest[5 + NB]
            sems = rest[6 + NB:6 + 2 * NB]
            scs = rest[6 + 2 * NB:6 + 3 * NB]
            msgvr = rest[6 + 3 * NB:6 + 4 * NB]
            stage, sums_sh = rest[6 + 4 * NB:8 + 4 * NB]
            sms = rest[8 + 4 * NB:8 + 5 * NB]
            smg = rest[8 + 5 * NB:8 + 6 * NB]
        else:
            (acc_hbm, sidx, didx) = rest[:3]
            bufs = rest[3:3 + NB]
            acc_sh = rest[3 + NB]
            sems = rest[4 + NB:4 + 2 * NB]
            scs = rest[4 + 2 * NB:4 + 3 * NB]
        buf0 = bufs[0]
        cc = lax.axis_index("c")
        ss = lax.axis_index("s")
        pltpu.sync_copy(src_hbm.at[cc, pl.ds(ss * CPT, CPT)], sidx)
        pltpu.sync_copy(dst_hbm.at[pl.ds(ss * CPT, CPT)], didx)
        for z in range(ZCH):
            pltpu.sync_copy(z_hbm.at[ss, pl.ds(z * ZR, ZR)], buf0)
            pltpu.sync_copy(buf0,
                            acc_sh.at[pl.ds(ss * NPT1 + z * ZR, ZR)])
        if msg is not None:
            zerof = jnp.zeros((LN,), _F32)
            for i in range(NPT1 // LN):
                stage[pl.ds(i * LN, LN)] = zerof
            pltpu.sync_copy(stage, sums_sh.at[pl.ds(ss * NPT1, NPT1)])
        plsc.subcore_barrier()

        def start(c, b):
            pltpu.async_copy(xs_hbm.at[sidx.at[c]], bufs[b], sems[b])
            if msg is not None:
                @pl.when(cc == 0)
                def _():
                    pltpu.async_copy(
                        msg_hbm.at[pl.ds((ss * CPT + c) * K, K)],
                        msgvr[b], smg[b])

        def wait_scatter(c, b):
            pltpu.make_async_copy(bufs[b], acc_sh.at[didx.at[c]],
                                  scs[b]).wait()
            if msg is not None:
                @pl.when(cc == 0)
                def _():
                    pltpu.make_async_copy(msgvr[b],
                                          sums_sh.at[didx.at[c]],
                                          sms[b]).wait()

        start(0, 0)
        start(1, 1)

        def chunk(i, _):
            for b in range(NB):
                c = i * NB + b
                pltpu.make_async_copy(xs_hbm.at[sidx.at[c]], bufs[b],
                                      sems[b]).wait()

                @pl.when(c >= 2)
                def _():
                    wait_scatter(c - 2, (b - 2) % NB)

                @pl.when(c + 2 < CPT)
                def _():
                    start(c + 2, (b + 2) % NB)

                pltpu.async_copy(bufs[b], acc_sh.at[didx.at[c]], scs[b],
                                 add=True)
                if msg is not None:
                    @pl.when(cc == 0)
                    def _():
                        pltpu.make_async_copy(
                            msg_hbm.at[pl.ds((ss * CPT + c) * K, K)],
                            msgvr[b], smg[b]).wait()
                        pltpu.async_copy(msgvr[b],
                                         sums_sh.at[didx.at[c]],
                                         sms[b], add=True)
            return 0

        lax.fori_loop(0, CPT // NB, chunk, 0)
        wait_scatter(CPT - 2, (CPT - 2) % NB)
        wait_scatter(CPT - 1, (CPT - 1) % NB)
        plsc.subcore_barrier()
        for z in range(ZCH):
            pltpu.sync_copy(acc_sh.at[pl.ds(ss * NPT1 + z * ZR, ZR)], buf0)
            pltpu.sync_copy(buf0, acc_hbm.at[cc, ss, pl.ds(z * ZR, ZR)])
        if msg is not None:
            @pl.when(cc == 0)
            def _():
                pltpu.sync_copy(sums_sh.at[pl.ds(ss * NPT1, NPT1)], stage)
                pltpu.sync_copy(stage, sum_hbm.at[ss])

    if msg is not None:
        return body(xst, srco, dstp, zc, msg)
    return body(xst, srco, dstp, zc)


def kernel(x, edge_index, mlp_W1, mlp_b1, mlp_W2, mlp_b2,
           low_W0, high_W0, low_W1, high_W1, out_W, out_b):
    N, C = x.shape
    H = C // NC
    E = edge_index.shape[1]
    N1 = N + PADN
    cpt = -(-E // (NS * K * 8)) * 8
    epad = NS * K * cpt
    pad = epad - E
    src = jnp.concatenate([edge_index[0], jnp.full((pad,), N, jnp.int32)])
    dst = jnp.concatenate([edge_index[1], jnp.full((pad,), N, jnp.int32)])
    off = jnp.arange(NC, dtype=jnp.int32)[:, None, None] * N1
    src2o = src.reshape(1, -1, K) + off
    dst2o = dst.reshape(1, -1, K) + off
    dst2p = dst.reshape(-1, K)

    Ast, Bst = _tc_ab(x, mlp_W1, mlp_b1)
    w2st = mlp_W2[:, 0].reshape(NC, H)
    cnt_parts, P = _sc_edge_mlp(Ast.reshape(NC * N1, H),
                                Bst.reshape(NC * N1, H), src2o, dst2o, w2st)
    cnt, dinv = _tc_deg(cnt_parts.reshape(-1, 1), x)
    msg = _tc_msg(P.reshape(NC, -1, K * LN), mlp_b2)

    zc = jnp.zeros((NS, NPT1, H), _F32)

    xs0st = _tc_xs(dinv, x)
    acc0, sums_parts = _sc_prop(xs0st.reshape(NC * N1, H), src2o, dst2p, zc,
                                msg=msg.reshape(-1))
    lh, x1 = _tc_layer0(acc0.reshape(NC, -1, H), sums_parts.reshape(-1, 1),
                        cnt, x, dinv, low_W0, high_W0)
    xs1st = _tc_xs(dinv, x1)
    acc1 = _sc_prop(xs1st.reshape(NC * N1, H), src2o, dst2p, zc)[0]
    return _tc_final(acc1.reshape(NC, -1, H), x1, lh, dinv,
                     low_W1, high_W1, out_W, out_b)
